# baseline (speedup 1.0000x reference)
import functools

import jax
import jax.numpy as jnp
from jax import lax
from jax.experimental import pallas as pl
from jax.experimental.pallas import tpu as pltpu
from jax.experimental.pallas import tpu_sc as plsc

VOCAB = 1000
MAIN = 896
TAIL_OFF = VOCAB - 128
N_EMBD = 64
NUM_WORKERS = 32
CHUNK = 32


def _table_kernel(tok_ref, w_ref, b_ref, main_ref, tail_ref):
    scores = (
        jnp.dot(tok_ref[...], w_ref[...], preferred_element_type=jnp.float32)
        + b_ref[...]
    )
    main_ref[...] = scores[:, :MAIN]
    tail_ref[...] = scores[:, TAIL_OFF:]


def _score_tables(tok_table, W, b):
    return pl.pallas_call(
        _table_kernel,
        out_shape=(
            jax.ShapeDtypeStruct((VOCAB, MAIN), jnp.float32),
            jax.ShapeDtypeStruct((VOCAB, 128), jnp.float32),
        ),
    )(tok_table, W, b.reshape(1, VOCAB))


_TAIL_SEGS = [(24 + 16 * k, MAIN + 16 * k) for k in range(6)] + [(112, 984)]


@functools.lru_cache(maxsize=None)
def _make_gather(n_rows):
    assert n_rows % (NUM_WORKERS * CHUNK) == 0
    rows_per_w = n_rows // NUM_WORKERS
    n_chunks = rows_per_w // CHUNK
    mesh = plsc.VectorSubcoreMesh(core_axis_name="c", subcore_axis_name="s")

    assert n_chunks % 2 == 0
    n_iters = n_chunks // 2

    assert CHUNK % 8 == 0
    batches_per_chunk = CHUNK // 8

    @functools.partial(
        pl.kernel,
        out_type=jax.ShapeDtypeStruct((n_rows // 8, 8, VOCAB), jnp.float32),
        mesh=mesh,
        scratch_types=[
            pltpu.VMEM((n_chunks, CHUNK), jnp.int32),
            pltpu.VMEM((CHUNK, VOCAB), jnp.float32),
            pltpu.VMEM((CHUNK, VOCAB), jnp.float32),
            pltpu.VMEM((CHUNK, 128), jnp.float32),
            pltpu.VMEM((CHUNK, 128), jnp.float32),
            pltpu.SemaphoreType.DMA,
            pltpu.SemaphoreType.DMA,
            pltpu.SemaphoreType.DMA,
            pltpu.SemaphoreType.DMA,
            pltpu.SemaphoreType.DMA,
            pltpu.SemaphoreType.DMA,
        ],
        compiler_params=pltpu.CompilerParams(needs_layout_passes=False),
    )
    def gather(main_hbm, tail_hbm, idx_hbm, out_hbm, idx_v,
               rows_a, rows_b, tail_a, tail_b,
               sem_ga, sem_gb, sem_ta, sem_tb, sem_wa, sem_wb):
        wid = lax.axis_index("s") * 2 + lax.axis_index("c")
        base = wid * rows_per_w

        def start_gather(c, rows_v, tail_v, sem, sem_t):
            ix = idx_v.at[c]
            pltpu.async_copy(main_hbm.at[ix], rows_v.at[:, pl.ds(0, MAIN)], sem)
            pltpu.async_copy(tail_hbm.at[ix], tail_v, sem_t)

        def wait_gather(c, rows_v, tail_v, sem, sem_t):
            ix = idx_v.at[c]
            pltpu.make_async_copy(main_hbm.at[ix], rows_v.at[:, pl.ds(0, MAIN)], sem).wait()
            pltpu.make_async_copy(tail_hbm.at[ix], tail_v, sem_t).wait()

        def fill_tail(rows_v, tail_v):
            lane = lax.iota(jnp.int32, 16)

            def row_body(r, carry):
                rvec = jnp.full((16,), r, dtype=jnp.int32)
                for src, dst in _TAIL_SEGS:
                    v = plsc.load_gather(tail_v, [rvec, src + lane])
                    plsc.store_scatter(rows_v, [rvec, dst + lane], v)
                return carry
            lax.fori_loop(0, CHUNK, row_body, 0)

        def start_write(c, rows_v, sem):
            b0 = (base + c * CHUNK) // 8
            for k in range(batches_per_chunk):
                pltpu.async_copy(rows_v.at[pl.ds(8 * k, 8)], out_hbm.at[b0 + k], sem)

        def wait_write(c, rows_v, sem):
            b0 = (base + c * CHUNK) // 8
            for k in range(batches_per_chunk):
                pltpu.make_async_copy(
                    rows_v.at[pl.ds(8 * k, 8)], out_hbm.at[b0 + k], sem).wait()

        pltpu.sync_copy(idx_hbm.at[wid], idx_v)
        start_gather(0, rows_a, tail_a, sem_ga, sem_ta)

        def body(g, carry):
            c0 = 2 * g

            @pl.when(g > 0)
            def _():
                wait_write(c0 - 1, rows_b, sem_wb)

            start_gather(c0 + 1, rows_b, tail_b, sem_gb, sem_tb)
            wait_gather(c0, rows_a, tail_a, sem_ga, sem_ta)
            fill_tail(rows_a, tail_a)
            start_write(c0, rows_a, sem_wa)

            @pl.when(g < n_iters - 1)
            def _():
                wait_write(c0, rows_a, sem_wa)
                start_gather(c0 + 2, rows_a, tail_a, sem_ga, sem_ta)

            wait_gather(c0 + 1, rows_b, tail_b, sem_gb, sem_tb)
            fill_tail(rows_b, tail_b)
            start_write(c0 + 1, rows_b, sem_wb)
            return carry

        lax.fori_loop(0, n_iters, body, 0)
        wait_write(n_chunks - 2, rows_a, sem_wa)
        wait_write(n_chunks - 1, rows_b, sem_wb)

    return gather


def kernel(idx, tok_table, pos_table, W, b):
    B, T = idx.shape
    n_rows = B * T
    t_main, t_tail = _score_tables(tok_table, W, b)
    flat = idx.reshape(NUM_WORKERS, n_rows // (NUM_WORKERS * CHUNK), CHUNK)
    flat = flat.astype(jnp.int32)
    out = _make_gather(n_rows)(t_main, t_tail, flat)
    assert out.shape == (B, T, VOCAB)
    return out


# speedup vs baseline: 1.5985x; 1.5985x over previous
import functools

import jax
import jax.numpy as jnp
from jax import lax
from jax.experimental import pallas as pl
from jax.experimental.pallas import tpu as pltpu
from jax.experimental.pallas import tpu_sc as plsc

VOCAB = 1000
N_EMBD = 64
EMBD_PAD = 128
NUM_WORKERS = 32
CHUNK = 128
BTILE = 256


@functools.lru_cache(maxsize=None)
def _make_emb_gather(T, B):
    n_tok = T * B
    assert n_tok % (NUM_WORKERS * CHUNK) == 0
    tok_per_w = n_tok // NUM_WORKERS
    n_chunks = tok_per_w // CHUNK
    assert B % tok_per_w == 0
    mesh = plsc.VectorSubcoreMesh(core_axis_name="c", subcore_axis_name="s")

    assert n_chunks % 2 == 0
    n_iters = n_chunks // 2

    @functools.partial(
        pl.kernel,
        out_type=jax.ShapeDtypeStruct((T, B, EMBD_PAD), jnp.float32),
        mesh=mesh,
        scratch_types=[
            pltpu.VMEM((n_chunks, CHUNK), jnp.int32),
            pltpu.VMEM((CHUNK, EMBD_PAD), jnp.float32),
            pltpu.VMEM((CHUNK, EMBD_PAD), jnp.float32),
            pltpu.SemaphoreType.DMA,
            pltpu.SemaphoreType.DMA,
            pltpu.SemaphoreType.DMA,
            pltpu.SemaphoreType.DMA,
        ],
        compiler_params=pltpu.CompilerParams(needs_layout_passes=False),
    )
    def gather(table_hbm, idx_hbm, out_hbm, idx_v, rows_a, rows_b,
               sem_ga, sem_gb, sem_wa, sem_wb):
        wid = lax.axis_index("s") * 2 + lax.axis_index("c")
        t = wid // (B // tok_per_w)
        b_base = (wid * tok_per_w) % B

        def start_gather(c, rows_v, sem):
            pltpu.async_copy(table_hbm.at[idx_v.at[c]], rows_v, sem)

        def wait_gather(c, rows_v, sem):
            pltpu.make_async_copy(table_hbm.at[idx_v.at[c]], rows_v, sem).wait()

        def start_write(c, rows_v, sem):
            pltpu.async_copy(
                rows_v, out_hbm.at[t, pl.ds(b_base + c * CHUNK, CHUNK)], sem)

        def wait_write(c, rows_v, sem):
            pltpu.make_async_copy(
                rows_v, out_hbm.at[t, pl.ds(b_base + c * CHUNK, CHUNK)], sem).wait()

        pltpu.sync_copy(idx_hbm.at[wid], idx_v)
        start_gather(0, rows_a, sem_ga)

        def body(g, carry):
            c0 = 2 * g

            @pl.when(g > 0)
            def _():
                wait_write(c0 - 1, rows_b, sem_wb)

            start_gather(c0 + 1, rows_b, sem_gb)
            wait_gather(c0, rows_a, sem_ga)
            start_write(c0, rows_a, sem_wa)

            @pl.when(g < n_iters - 1)
            def _():
                wait_write(c0, rows_a, sem_wa)
                start_gather(c0 + 2, rows_a, sem_ga)

            wait_gather(c0 + 1, rows_b, sem_gb)
            start_write(c0 + 1, rows_b, sem_wb)
            return carry

        lax.fori_loop(0, n_iters, body, 0)
        wait_write(n_chunks - 2, rows_a, sem_wa)
        wait_write(n_chunks - 1, rows_b, sem_wb)

    return gather


def _matmul_body(emb_ref, w_ref, b_ref, out_ref):
    e = emb_ref[0, :, :N_EMBD]
    s = lax.dot_general(
        w_ref[...], e, (((0,), (1,)), ((), ())),
        preferred_element_type=jnp.float32)
    out_ref[0] = s + b_ref[...]


@functools.lru_cache(maxsize=None)
def _make_matmul(T, B):
    assert B % BTILE == 0
    return pl.pallas_call(
        _matmul_body,
        grid=(T, B // BTILE),
        in_specs=[
            pl.BlockSpec((1, BTILE, EMBD_PAD), lambda t, bt: (t, bt, 0)),
            pl.BlockSpec((N_EMBD, VOCAB), lambda t, bt: (0, 0)),
            pl.BlockSpec((VOCAB, 1), lambda t, bt: (0, 0)),
        ],
        out_specs=pl.BlockSpec((1, VOCAB, BTILE), lambda t, bt: (t, 0, bt)),
        out_shape=jax.ShapeDtypeStruct((T, VOCAB, B), jnp.float32),
    )


def kernel(idx, tok_table, pos_table, W, b):
    B, T = idx.shape
    tok_pad = jnp.pad(tok_table, ((0, 0), (0, EMBD_PAD - N_EMBD)))
    idx_t = idx.T.reshape(NUM_WORKERS, (B * T) // (NUM_WORKERS * CHUNK), CHUNK)
    idx_t = idx_t.astype(jnp.int32)
    emb = _make_emb_gather(T, B)(tok_pad, idx_t)
    scores_t = _make_matmul(T, B)(emb, W, b.reshape(VOCAB, 1))
    return scores_t.transpose(2, 0, 1)


# speedup vs baseline: 2.8349x; 1.7735x over previous
import functools

import jax
import jax.numpy as jnp
from jax import lax
from jax.experimental import pallas as pl
from jax.experimental.pallas import tpu as pltpu
from jax.experimental.pallas import tpu_sc as plsc

VOCAB = 1000
N_EMBD = 64
EMBD_PAD = 128
NUM_WORKERS = 32
CHUNK = 128
BTILE = 1024


@functools.lru_cache(maxsize=None)
def _make_emb_gather(T, B):
    n_tok = T * B
    assert n_tok % (NUM_WORKERS * CHUNK) == 0
    tok_per_w = n_tok // NUM_WORKERS
    n_chunks = tok_per_w // CHUNK
    assert B % tok_per_w == 0
    mesh = plsc.VectorSubcoreMesh(core_axis_name="c", subcore_axis_name="s")

    assert n_chunks % 2 == 0
    n_iters = n_chunks // 2

    @functools.partial(
        pl.kernel,
        out_type=jax.ShapeDtypeStruct((T, B, EMBD_PAD), jnp.float32),
        mesh=mesh,
        scratch_types=[
            pltpu.VMEM((n_chunks, CHUNK), jnp.int32),
            pltpu.VMEM((CHUNK, EMBD_PAD), jnp.float32),
            pltpu.VMEM((CHUNK, EMBD_PAD), jnp.float32),
            pltpu.SemaphoreType.DMA,
            pltpu.SemaphoreType.DMA,
            pltpu.SemaphoreType.DMA,
            pltpu.SemaphoreType.DMA,
        ],
        compiler_params=pltpu.CompilerParams(needs_layout_passes=False),
    )
    def gather(table_hbm, idx_hbm, out_hbm, idx_v, rows_a, rows_b,
               sem_ga, sem_gb, sem_wa, sem_wb):
        wid = lax.axis_index("s") * 2 + lax.axis_index("c")
        t = wid // (B // tok_per_w)
        b_base = (wid * tok_per_w) % B

        def start_gather(c, rows_v, sem):
            pltpu.async_copy(table_hbm.at[idx_v.at[c]], rows_v, sem)

        def wait_gather(c, rows_v, sem):
            pltpu.make_async_copy(table_hbm.at[idx_v.at[c]], rows_v, sem).wait()

        def start_write(c, rows_v, sem):
            pltpu.async_copy(
                rows_v, out_hbm.at[t, pl.ds(b_base + c * CHUNK, CHUNK)], sem)

        def wait_write(c, rows_v, sem):
            pltpu.make_async_copy(
                rows_v, out_hbm.at[t, pl.ds(b_base + c * CHUNK, CHUNK)], sem).wait()

        pltpu.sync_copy(idx_hbm.at[wid], idx_v)
        start_gather(0, rows_a, sem_ga)

        def body(g, carry):
            c0 = 2 * g

            @pl.when(g > 0)
            def _():
                wait_write(c0 - 1, rows_b, sem_wb)

            start_gather(c0 + 1, rows_b, sem_gb)
            wait_gather(c0, rows_a, sem_ga)
            start_write(c0, rows_a, sem_wa)

            @pl.when(g < n_iters - 1)
            def _():
                wait_write(c0, rows_a, sem_wa)
                start_gather(c0 + 2, rows_a, sem_ga)

            wait_gather(c0 + 1, rows_b, sem_gb)
            start_write(c0 + 1, rows_b, sem_wb)
            return carry

        lax.fori_loop(0, n_iters, body, 0)
        wait_write(n_chunks - 2, rows_a, sem_wa)
        wait_write(n_chunks - 1, rows_b, sem_wb)

    return gather


def _matmul_body(emb_ref, w_ref, b_ref, out_ref):
    e = emb_ref[0, :, :N_EMBD]
    s = lax.dot_general(
        w_ref[...], e, (((0,), (1,)), ((), ())),
        preferred_element_type=jnp.float32)
    out_ref[0] = s + b_ref[...]


@functools.lru_cache(maxsize=None)
def _make_matmul(T, B):
    assert B % BTILE == 0
    return pl.pallas_call(
        _matmul_body,
        grid=(T, B // BTILE),
        in_specs=[
            pl.BlockSpec((1, BTILE, EMBD_PAD), lambda t, bt: (t, bt, 0)),
            pl.BlockSpec((N_EMBD, VOCAB), lambda t, bt: (0, 0)),
            pl.BlockSpec((VOCAB, 1), lambda t, bt: (0, 0)),
        ],
        out_specs=pl.BlockSpec((1, VOCAB, BTILE), lambda t, bt: (t, 0, bt)),
        out_shape=jax.ShapeDtypeStruct((T, VOCAB, B), jnp.float32),
    )


def kernel(idx, tok_table, pos_table, W, b):
    B, T = idx.shape
    tok_pad = jnp.pad(tok_table, ((0, 0), (0, EMBD_PAD - N_EMBD)))
    idx_t = idx.T.reshape(NUM_WORKERS, (B * T) // (NUM_WORKERS * CHUNK), CHUNK)
    idx_t = idx_t.astype(jnp.int32)
    emb = _make_emb_gather(T, B)(tok_pad, idx_t)
    scores_t = _make_matmul(T, B)(emb, W, b.reshape(VOCAB, 1))
    return scores_t.transpose(2, 0, 1)


# speedup vs baseline: 3.1471x; 1.1101x over previous
import functools

import jax
import jax.numpy as jnp
from jax import lax
from jax.experimental import pallas as pl
from jax.experimental.pallas import tpu as pltpu
from jax.experimental.pallas import tpu_sc as plsc

VOCAB = 1000
N_EMBD = 64
EMBD_PAD = 128
NUM_WORKERS = 32
CHUNK = 128
BTILE = 2048


@functools.lru_cache(maxsize=None)
def _make_emb_gather(T, B):
    n_tok = T * B
    assert n_tok % (NUM_WORKERS * CHUNK) == 0
    tok_per_w = n_tok // NUM_WORKERS
    n_chunks = tok_per_w // CHUNK
    assert B % tok_per_w == 0
    mesh = plsc.VectorSubcoreMesh(core_axis_name="c", subcore_axis_name="s")

    assert n_chunks % 2 == 0
    n_iters = n_chunks // 2

    @functools.partial(
        pl.kernel,
        out_type=jax.ShapeDtypeStruct((T, B, EMBD_PAD), jnp.float32),
        mesh=mesh,
        scratch_types=[
            pltpu.VMEM((n_chunks, CHUNK), jnp.int32),
            pltpu.VMEM((CHUNK, EMBD_PAD), jnp.float32),
            pltpu.VMEM((CHUNK, EMBD_PAD), jnp.float32),
            pltpu.SemaphoreType.DMA,
            pltpu.SemaphoreType.DMA,
            pltpu.SemaphoreType.DMA,
            pltpu.SemaphoreType.DMA,
        ],
        compiler_params=pltpu.CompilerParams(needs_layout_passes=False),
    )
    def gather(table_hbm, idx_hbm, out_hbm, idx_v, rows_a, rows_b,
               sem_ga, sem_gb, sem_wa, sem_wb):
        wid = lax.axis_index("s") * 2 + lax.axis_index("c")
        t = wid // (B // tok_per_w)
        b_base = (wid * tok_per_w) % B

        def start_gather(c, rows_v, sem):
            pltpu.async_copy(table_hbm.at[idx_v.at[c]], rows_v, sem)

        def wait_gather(c, rows_v, sem):
            pltpu.make_async_copy(table_hbm.at[idx_v.at[c]], rows_v, sem).wait()

        def start_write(c, rows_v, sem):
            pltpu.async_copy(
                rows_v, out_hbm.at[t, pl.ds(b_base + c * CHUNK, CHUNK)], sem)

        def wait_write(c, rows_v, sem):
            pltpu.make_async_copy(
                rows_v, out_hbm.at[t, pl.ds(b_base + c * CHUNK, CHUNK)], sem).wait()

        pltpu.sync_copy(idx_hbm.at[wid], idx_v)
        start_gather(0, rows_a, sem_ga)

        def body(g, carry):
            c0 = 2 * g

            @pl.when(g > 0)
            def _():
                wait_write(c0 - 1, rows_b, sem_wb)

            start_gather(c0 + 1, rows_b, sem_gb)
            wait_gather(c0, rows_a, sem_ga)
            start_write(c0, rows_a, sem_wa)

            @pl.when(g < n_iters - 1)
            def _():
                wait_write(c0, rows_a, sem_wa)
                start_gather(c0 + 2, rows_a, sem_ga)

            wait_gather(c0 + 1, rows_b, sem_gb)
            start_write(c0 + 1, rows_b, sem_wb)
            return carry

        lax.fori_loop(0, n_iters, body, 0)
        wait_write(n_chunks - 2, rows_a, sem_wa)
        wait_write(n_chunks - 1, rows_b, sem_wb)

    return gather


def _matmul_body(emb_ref, w_ref, b_ref, out_ref):
    e = emb_ref[0, :, :N_EMBD]
    s = lax.dot_general(
        w_ref[...], e, (((0,), (1,)), ((), ())),
        preferred_element_type=jnp.float32)
    out_ref[0] = s + b_ref[...]


@functools.lru_cache(maxsize=None)
def _make_matmul(T, B):
    assert B % BTILE == 0
    return pl.pallas_call(
        _matmul_body,
        grid=(T, B // BTILE),
        in_specs=[
            pl.BlockSpec((1, BTILE, EMBD_PAD), lambda t, bt: (t, bt, 0)),
            pl.BlockSpec((N_EMBD, VOCAB), lambda t, bt: (0, 0)),
            pl.BlockSpec((VOCAB, 1), lambda t, bt: (0, 0)),
        ],
        out_specs=pl.BlockSpec((1, VOCAB, BTILE), lambda t, bt: (t, 0, bt)),
        out_shape=jax.ShapeDtypeStruct((T, VOCAB, B), jnp.float32),
    )


def kernel(idx, tok_table, pos_table, W, b):
    B, T = idx.shape
    tok_pad = jnp.pad(tok_table, ((0, 0), (0, EMBD_PAD - N_EMBD)))
    idx_t = idx.T.reshape(NUM_WORKERS, (B * T) // (NUM_WORKERS * CHUNK), CHUNK)
    idx_t = idx_t.astype(jnp.int32)
    emb = _make_emb_gather(T, B)(tok_pad, idx_t)
    scores_t = _make_matmul(T, B)(emb, W, b.reshape(VOCAB, 1))
    return scores_t.transpose(2, 0, 1)


# speedup vs baseline: 3.2136x; 1.0211x over previous
import functools

import jax
import jax.numpy as jnp
from jax import lax
from jax.experimental import pallas as pl
from jax.experimental.pallas import tpu as pltpu
from jax.experimental.pallas import tpu_sc as plsc

VOCAB = 1000
N_EMBD = 64
EMBD_PAD = 128
NUM_WORKERS = 32
CHUNK = 128
BTILE = 4096


@functools.lru_cache(maxsize=None)
def _make_emb_gather(T, B):
    n_tok = T * B
    assert n_tok % (NUM_WORKERS * CHUNK) == 0
    tok_per_w = n_tok // NUM_WORKERS
    n_chunks = tok_per_w // CHUNK
    assert B % tok_per_w == 0
    mesh = plsc.VectorSubcoreMesh(core_axis_name="c", subcore_axis_name="s")

    assert n_chunks % 2 == 0
    n_iters = n_chunks // 2

    @functools.partial(
        pl.kernel,
        out_type=jax.ShapeDtypeStruct((T, B, EMBD_PAD), jnp.float32),
        mesh=mesh,
        scratch_types=[
            pltpu.VMEM((n_chunks, CHUNK), jnp.int32),
            pltpu.VMEM((CHUNK, EMBD_PAD), jnp.float32),
            pltpu.VMEM((CHUNK, EMBD_PAD), jnp.float32),
            pltpu.SemaphoreType.DMA,
            pltpu.SemaphoreType.DMA,
            pltpu.SemaphoreType.DMA,
            pltpu.SemaphoreType.DMA,
        ],
        compiler_params=pltpu.CompilerParams(needs_layout_passes=False),
    )
    def gather(table_hbm, idx_hbm, out_hbm, idx_v, rows_a, rows_b,
               sem_ga, sem_gb, sem_wa, sem_wb):
        wid = lax.axis_index("s") * 2 + lax.axis_index("c")
        t = wid // (B // tok_per_w)
        b_base = (wid * tok_per_w) % B

        def start_gather(c, rows_v, sem):
            pltpu.async_copy(table_hbm.at[idx_v.at[c]], rows_v, sem)

        def wait_gather(c, rows_v, sem):
            pltpu.make_async_copy(table_hbm.at[idx_v.at[c]], rows_v, sem).wait()

        def start_write(c, rows_v, sem):
            pltpu.async_copy(
                rows_v, out_hbm.at[t, pl.ds(b_base + c * CHUNK, CHUNK)], sem)

        def wait_write(c, rows_v, sem):
            pltpu.make_async_copy(
                rows_v, out_hbm.at[t, pl.ds(b_base + c * CHUNK, CHUNK)], sem).wait()

        pltpu.sync_copy(idx_hbm.at[wid], idx_v)
        start_gather(0, rows_a, sem_ga)

        def body(g, carry):
            c0 = 2 * g

            @pl.when(g > 0)
            def _():
                wait_write(c0 - 1, rows_b, sem_wb)

            start_gather(c0 + 1, rows_b, sem_gb)
            wait_gather(c0, rows_a, sem_ga)
            start_write(c0, rows_a, sem_wa)

            @pl.when(g < n_iters - 1)
            def _():
                wait_write(c0, rows_a, sem_wa)
                start_gather(c0 + 2, rows_a, sem_ga)

            wait_gather(c0 + 1, rows_b, sem_gb)
            start_write(c0 + 1, rows_b, sem_wb)
            return carry

        lax.fori_loop(0, n_iters, body, 0)
        wait_write(n_chunks - 2, rows_a, sem_wa)
        wait_write(n_chunks - 1, rows_b, sem_wb)

    return gather


def _matmul_body(emb_ref, w_ref, b_ref, out_ref):
    e = emb_ref[0, :, :N_EMBD]
    s = lax.dot_general(
        w_ref[...], e, (((0,), (1,)), ((), ())),
        preferred_element_type=jnp.float32)
    out_ref[0] = s + b_ref[...]


@functools.lru_cache(maxsize=None)
def _make_matmul(T, B):
    assert B % BTILE == 0
    return pl.pallas_call(
        _matmul_body,
        grid=(T, B // BTILE),
        in_specs=[
            pl.BlockSpec((1, BTILE, EMBD_PAD), lambda t, bt: (t, bt, 0)),
            pl.BlockSpec((N_EMBD, VOCAB), lambda t, bt: (0, 0)),
            pl.BlockSpec((VOCAB, 1), lambda t, bt: (0, 0)),
        ],
        out_specs=pl.BlockSpec((1, VOCAB, BTILE), lambda t, bt: (t, 0, bt)),
        out_shape=jax.ShapeDtypeStruct((T, VOCAB, B), jnp.float32),
    )


def kernel(idx, tok_table, pos_table, W, b):
    B, T = idx.shape
    tok_pad = jnp.pad(tok_table, ((0, 0), (0, EMBD_PAD - N_EMBD)))
    idx_t = idx.T.reshape(NUM_WORKERS, (B * T) // (NUM_WORKERS * CHUNK), CHUNK)
    idx_t = idx_t.astype(jnp.int32)
    emb = _make_emb_gather(T, B)(tok_pad, idx_t)
    scores_t = _make_matmul(T, B)(emb, W, b.reshape(VOCAB, 1))
    return scores_t.transpose(2, 0, 1)
